# baseline (speedup 1.0000x reference)
import functools

import jax
import jax.numpy as jnp
from jax import lax
from jax.experimental import pallas as pl
from jax.experimental.pallas import tpu as pltpu
from jax.experimental.pallas import tpu_sc as plsc

B, C, H, W = 32, 7, 64, 64
GH, GW = 512, 512
BLK = 8
HW = H * W
NC, NS, L = 2, 16, 16
CHUNK = 32
NCHUNK = GH // CHUNK
BR_PER_CHUNK = CHUNK // BLK
VPR = GW // L
THRESH = (1, 2, 4, 8, 16, 32, 49)


def _sc_cls_body(gt_hbm, cls_hbm, in_buf0, in_buf1, rowsum, out_buf,
                 sem0, sem1):
    wid = lax.axis_index("s") * NC + lax.axis_index("c")
    lanes = lax.iota(jnp.int32, L)
    bufs = (in_buf0, in_buf1)
    sems = (sem0, sem1)

    def src_rows(chunk):
        return gt_hbm.at[wid, 0, pl.ds(chunk * CHUNK, CHUNK)]

    def compute_chunk(chunk, buf):
        def br_body(br, c2):
            for v in range(VPR):
                s = buf[br * BLK, pl.ds(v * L, L)]
                for r in range(1, BLK):
                    s = s + buf[br * BLK + r, pl.ds(v * L, L)]
                rowsum[pl.ds(v * L, L)] = s
            br_glob = chunk * BR_PER_CHUNK + br
            orow = br_glob // 2
            ocol0 = (br_glob % 2) * W
            for g in range(W // L):
                idx0 = lanes * BLK + g * (L * BLK)
                acc = plsc.load_gather(rowsum, [idx0])
                for j in range(1, BLK):
                    acc = acc + plsc.load_gather(rowsum, [idx0 + j])
                cls = (acc >= THRESH[0]).astype(jnp.int32)
                for t in THRESH[1:]:
                    cls = cls + (acc >= t).astype(jnp.int32)
                out_buf[orow, pl.ds(ocol0 + g * L, L)] = cls
            return c2

        lax.fori_loop(0, BR_PER_CHUNK, br_body, 0)

    pltpu.async_copy(src_rows(0), bufs[0], sems[0])
    pltpu.async_copy(src_rows(1), bufs[1], sems[1])

    def chunk_pair(i, carry):
        for b in range(2):
            chunk = i * 2 + b
            pltpu.make_async_copy(src_rows(0), bufs[b], sems[b]).wait()
            compute_chunk(chunk, bufs[b])
            pltpu.async_copy(src_rows(chunk + 2), bufs[b], sems[b])
        return carry

    lax.fori_loop(0, NCHUNK // 2 - 1, chunk_pair, 0)
    for b in range(2):
        pltpu.make_async_copy(src_rows(0), bufs[b], sems[b]).wait()
        compute_chunk(NCHUNK - 2 + b, bufs[b])
    pltpu.sync_copy(out_buf, cls_hbm.at[wid])


@jax.jit
def _sc_cls(gt4):
    mesh = plsc.VectorSubcoreMesh(core_axis_name="c", subcore_axis_name="s",
                                  num_cores=NC, num_subcores=NS)
    return pl.kernel(
        _sc_cls_body,
        out_type=jax.ShapeDtypeStruct((B, HW // 128, 128), jnp.int32),
        mesh=mesh,
        compiler_params=pltpu.CompilerParams(needs_layout_passes=False),
        scratch_types=[
            pltpu.VMEM((CHUNK, GW), jnp.int32),
            pltpu.VMEM((CHUNK, GW), jnp.int32),
            pltpu.VMEM((GW,), jnp.int32),
            pltpu.VMEM((HW // 128, 128), jnp.int32),
            pltpu.SemaphoreType.DMA,
            pltpu.SemaphoreType.DMA,
        ],
    )(gt4)


IB = 4
PR = HW // 128


def _tc_loss_body(logits_ref, cls_ref, out_ref):
    x = logits_ref[...]
    s = jnp.sum(jnp.exp(x), axis=1, keepdims=True)
    lse = jnp.log(s)
    cls = cls_ref[...][:, None, :, :]
    tgt = cls - 1
    picked = jnp.zeros_like(lse)
    for cc in range(C):
        picked = picked + jnp.where(tgt == cc, x[:, cc:cc + 1, :, :], 0.0)
    contrib = jnp.sum(jnp.where(cls > 0, lse - picked, 0.0))

    @pl.when(pl.program_id(0) == 0)
    def _():
        out_ref[0, 0] = 0.0

    out_ref[0, 0] += contrib


@jax.jit
def _tc_loss(logits4, cls3):
    return pl.pallas_call(
        _tc_loss_body,
        grid=(B // IB,),
        in_specs=[
            pl.BlockSpec((IB, C, PR, 128), lambda b: (b, 0, 0, 0)),
            pl.BlockSpec((IB, PR, 128), lambda b: (b, 0, 0)),
        ],
        out_specs=pl.BlockSpec((1, 1), lambda b: (0, 0),
                               memory_space=pltpu.SMEM),
        out_shape=jax.ShapeDtypeStruct((1, 1), jnp.float32),
    )(logits4, cls3)


def kernel(logit_maps, gt_den_maps):
    cls = _sc_cls(gt_den_maps)
    logits4 = logit_maps.reshape(B, C, PR, 128)
    total = _tc_loss(logits4, cls)
    loss = total[0, 0] * jnp.float32(1.0 / B)
    return (loss, {"cls_zice": lax.stop_gradient(loss)})


# speedup vs baseline: 1.0658x; 1.0658x over previous
import functools

import jax
import jax.numpy as jnp
from jax import lax
from jax.experimental import pallas as pl
from jax.experimental.pallas import tpu as pltpu
from jax.experimental.pallas import tpu_sc as plsc

B, C, H, W = 32, 7, 64, 64
GH, GW = 512, 512
BLK = 8
HW = H * W
NC, NS, L = 2, 16, 16
CHUNK = 32
NCHUNK = GH // CHUNK
BR_PER_CHUNK = CHUNK // BLK
VPR = GW // L
THRESH = (1, 2, 4, 8, 16, 32, 49)


def _sc_cls_body(gt_hbm, cls_hbm, in_buf0, in_buf1, rowsum, out_buf,
                 sem0, sem1):
    wid = lax.axis_index("s") * NC + lax.axis_index("c")
    lanes = lax.iota(jnp.int32, L)
    bufs = (in_buf0, in_buf1)
    sems = (sem0, sem1)

    def src_rows(chunk):
        return gt_hbm.at[wid, 0, pl.ds(chunk * CHUNK, CHUNK)]

    def compute_chunk(chunk, buf):
        def br_body(br, c2):
            for v0 in range(0, VPR, 8):
                sums = []
                for v in range(v0, v0 + 8):
                    xs = [buf[br * BLK + r, pl.ds(v * L, L)]
                          for r in range(BLK)]
                    while len(xs) > 1:
                        xs = [xs[i] + xs[i + 1] for i in range(0, len(xs), 2)]
                    sums.append(xs[0])
                for dv, s in enumerate(sums):
                    rowsum[pl.ds((v0 + dv) * L, L)] = s
            br_glob = chunk * BR_PER_CHUNK + br
            orow = br_glob // 2
            ocol0 = (br_glob % 2) * W
            for g in range(W // L):
                idx0 = lanes * BLK + g * (L * BLK)
                gs = [plsc.load_gather(rowsum, [idx0 + j]) for j in range(BLK)]
                while len(gs) > 1:
                    gs = [gs[i] + gs[i + 1] for i in range(0, len(gs), 2)]
                acc = gs[0]
                cls = (acc >= THRESH[0]).astype(jnp.int32)
                for t in THRESH[1:]:
                    cls = cls + (acc >= t).astype(jnp.int32)
                out_buf[orow, pl.ds(ocol0 + g * L, L)] = cls
            return c2

        lax.fori_loop(0, BR_PER_CHUNK, br_body, 0)

    pltpu.async_copy(src_rows(0), bufs[0], sems[0])
    pltpu.async_copy(src_rows(1), bufs[1], sems[1])

    def chunk_pair(i, carry):
        for b in range(2):
            chunk = i * 2 + b
            pltpu.make_async_copy(src_rows(0), bufs[b], sems[b]).wait()
            compute_chunk(chunk, bufs[b])
            pltpu.async_copy(src_rows(chunk + 2), bufs[b], sems[b])
        return carry

    lax.fori_loop(0, NCHUNK // 2 - 1, chunk_pair, 0)
    for b in range(2):
        pltpu.make_async_copy(src_rows(0), bufs[b], sems[b]).wait()
        compute_chunk(NCHUNK - 2 + b, bufs[b])
    pltpu.sync_copy(out_buf, cls_hbm.at[wid])


@jax.jit
def _sc_cls(gt4):
    mesh = plsc.VectorSubcoreMesh(core_axis_name="c", subcore_axis_name="s",
                                  num_cores=NC, num_subcores=NS)
    return pl.kernel(
        _sc_cls_body,
        out_type=jax.ShapeDtypeStruct((B, HW // 128, 128), jnp.int32),
        mesh=mesh,
        compiler_params=pltpu.CompilerParams(needs_layout_passes=False),
        scratch_types=[
            pltpu.VMEM((CHUNK, GW), jnp.int32),
            pltpu.VMEM((CHUNK, GW), jnp.int32),
            pltpu.VMEM((GW,), jnp.int32),
            pltpu.VMEM((HW // 128, 128), jnp.int32),
            pltpu.SemaphoreType.DMA,
            pltpu.SemaphoreType.DMA,
        ],
    )(gt4)


IB = 4
PR = HW // 128


def _tc_loss_body(logits_ref, cls_ref, out_ref):
    x = logits_ref[...]
    s = jnp.sum(jnp.exp(x), axis=1, keepdims=True)
    lse = jnp.log(s)
    cls = cls_ref[...][:, None, :, :]
    tgt = cls - 1
    picked = jnp.zeros_like(lse)
    for cc in range(C):
        picked = picked + jnp.where(tgt == cc, x[:, cc:cc + 1, :, :], 0.0)
    contrib = jnp.sum(jnp.where(cls > 0, lse - picked, 0.0))

    @pl.when(pl.program_id(0) == 0)
    def _():
        out_ref[0, 0] = 0.0

    out_ref[0, 0] += contrib


@jax.jit
def _tc_loss(logits4, cls3):
    return pl.pallas_call(
        _tc_loss_body,
        grid=(B // IB,),
        in_specs=[
            pl.BlockSpec((IB, C, PR, 128), lambda b: (b, 0, 0, 0)),
            pl.BlockSpec((IB, PR, 128), lambda b: (b, 0, 0)),
        ],
        out_specs=pl.BlockSpec((1, 1), lambda b: (0, 0),
                               memory_space=pltpu.SMEM),
        out_shape=jax.ShapeDtypeStruct((1, 1), jnp.float32),
    )(logits4, cls3)


def kernel(logit_maps, gt_den_maps):
    cls = _sc_cls(gt_den_maps)
    logits4 = logit_maps.reshape(B, C, PR, 128)
    total = _tc_loss(logits4, cls)
    loss = total[0, 0] * jnp.float32(1.0 / B)
    return (loss, {"cls_zice": lax.stop_gradient(loss)})


# speedup vs baseline: 1.1191x; 1.0501x over previous
import functools

import jax
import jax.numpy as jnp
from jax import lax
from jax.experimental import pallas as pl
from jax.experimental.pallas import tpu as pltpu
from jax.experimental.pallas import tpu_sc as plsc

B, C, H, W = 32, 7, 64, 64
GH, GW = 512, 512
BLK = 8
HW = H * W
NC, NS, L = 2, 16, 16
CHUNK = 64
NCHUNK = GH // CHUNK
BR_PER_CHUNK = CHUNK // BLK
VPR = GW // L
THRESH = (1, 2, 4, 8, 16, 32, 49)


def _sc_cls_body(gt_hbm, cls_hbm, in_buf0, in_buf1, rowsum, out_buf,
                 sem0, sem1):
    wid = lax.axis_index("s") * NC + lax.axis_index("c")
    lanes = lax.iota(jnp.int32, L)
    bufs = (in_buf0, in_buf1)
    sems = (sem0, sem1)

    def src_rows(chunk):
        return gt_hbm.at[wid, 0, pl.ds(chunk * CHUNK, CHUNK)]

    def compute_chunk(chunk, buf):
        def br_body(br, c2):
            for v0 in range(0, VPR, 8):
                sums = []
                for v in range(v0, v0 + 8):
                    xs = [buf[br * BLK + r, pl.ds(v * L, L)]
                          for r in range(BLK)]
                    while len(xs) > 1:
                        xs = [xs[i] + xs[i + 1] for i in range(0, len(xs), 2)]
                    sums.append(xs[0])
                for dv, s in enumerate(sums):
                    rowsum[pl.ds((v0 + dv) * L, L)] = s
            br_glob = chunk * BR_PER_CHUNK + br
            orow = br_glob // 2
            ocol0 = (br_glob % 2) * W
            for g in range(W // L):
                idx0 = lanes * BLK + g * (L * BLK)
                gs = [plsc.load_gather(rowsum, [idx0 + j]) for j in range(BLK)]
                while len(gs) > 1:
                    gs = [gs[i] + gs[i + 1] for i in range(0, len(gs), 2)]
                acc = gs[0]
                cls = (acc >= THRESH[0]).astype(jnp.int32)
                for t in THRESH[1:]:
                    cls = cls + (acc >= t).astype(jnp.int32)
                out_buf[orow, pl.ds(ocol0 + g * L, L)] = cls
            return c2

        lax.fori_loop(0, BR_PER_CHUNK, br_body, 0)

    pltpu.async_copy(src_rows(0), bufs[0], sems[0])
    pltpu.async_copy(src_rows(1), bufs[1], sems[1])

    def chunk_pair(i, carry):
        for b in range(2):
            chunk = i * 2 + b
            pltpu.make_async_copy(src_rows(0), bufs[b], sems[b]).wait()
            compute_chunk(chunk, bufs[b])
            pltpu.async_copy(src_rows(chunk + 2), bufs[b], sems[b])
        return carry

    lax.fori_loop(0, NCHUNK // 2 - 1, chunk_pair, 0)
    for b in range(2):
        pltpu.make_async_copy(src_rows(0), bufs[b], sems[b]).wait()
        compute_chunk(NCHUNK - 2 + b, bufs[b])
    pltpu.sync_copy(out_buf, cls_hbm.at[wid])


@jax.jit
def _sc_cls(gt4):
    mesh = plsc.VectorSubcoreMesh(core_axis_name="c", subcore_axis_name="s",
                                  num_cores=NC, num_subcores=NS)
    return pl.kernel(
        _sc_cls_body,
        out_type=jax.ShapeDtypeStruct((B, HW // 128, 128), jnp.int32),
        mesh=mesh,
        compiler_params=pltpu.CompilerParams(needs_layout_passes=False),
        scratch_types=[
            pltpu.VMEM((CHUNK, GW), jnp.int32),
            pltpu.VMEM((CHUNK, GW), jnp.int32),
            pltpu.VMEM((GW,), jnp.int32),
            pltpu.VMEM((HW // 128, 128), jnp.int32),
            pltpu.SemaphoreType.DMA,
            pltpu.SemaphoreType.DMA,
        ],
    )(gt4)


IB = 8
PR = HW // 128


def _tc_loss_body(logits_ref, cls_ref, out_ref):
    x = logits_ref[...]
    s = jnp.sum(jnp.exp(x), axis=1, keepdims=True)
    lse = jnp.log(s)
    cls = cls_ref[...][:, None, :, :]
    tgt = cls - 1
    picked = jnp.zeros_like(lse)
    for cc in range(C):
        picked = picked + jnp.where(tgt == cc, x[:, cc:cc + 1, :, :], 0.0)
    contrib = jnp.sum(jnp.where(cls > 0, lse - picked, 0.0))

    @pl.when(pl.program_id(0) == 0)
    def _():
        out_ref[0, 0] = 0.0

    out_ref[0, 0] += contrib


@jax.jit
def _tc_loss(logits4, cls3):
    return pl.pallas_call(
        _tc_loss_body,
        grid=(B // IB,),
        in_specs=[
            pl.BlockSpec((IB, C, PR, 128), lambda b: (b, 0, 0, 0)),
            pl.BlockSpec((IB, PR, 128), lambda b: (b, 0, 0)),
        ],
        out_specs=pl.BlockSpec((1, 1), lambda b: (0, 0),
                               memory_space=pltpu.SMEM),
        out_shape=jax.ShapeDtypeStruct((1, 1), jnp.float32),
    )(logits4, cls3)


def kernel(logit_maps, gt_den_maps):
    cls = _sc_cls(gt_den_maps)
    logits4 = logit_maps.reshape(B, C, PR, 128)
    total = _tc_loss(logits4, cls)
    loss = total[0, 0] * jnp.float32(1.0 / B)
    return (loss, {"cls_zice": lax.stop_gradient(loss)})


# speedup vs baseline: 1.1255x; 1.0056x over previous
import functools

import jax
import jax.numpy as jnp
from jax import lax
from jax.experimental import pallas as pl
from jax.experimental.pallas import tpu as pltpu
from jax.experimental.pallas import tpu_sc as plsc

B, C, H, W = 32, 7, 64, 64
GH, GW = 512, 512
BLK = 8
HW = H * W
NC, NS, L = 2, 16, 16
CHUNK = 64
NCHUNK = GH // CHUNK
BR_PER_CHUNK = CHUNK // BLK
VPR = GW // L
THRESH = (1, 2, 4, 8, 16, 32, 49)


def _sc_cls_body(gt_hbm, cls_hbm, in_buf0, in_buf1, rowsum, out_buf,
                 sem0, sem1):
    wid = lax.axis_index("s") * NC + lax.axis_index("c")
    lanes = lax.iota(jnp.int32, L)
    bufs = (in_buf0, in_buf1)
    sems = (sem0, sem1)

    def src_rows(chunk):
        return gt_hbm.at[wid, 0, pl.ds(chunk * CHUNK, CHUNK)]

    def compute_chunk(chunk, buf):
        def br_body(br, c2):
            for v0 in range(0, VPR, 8):
                sums = []
                for v in range(v0, v0 + 8):
                    xs = [buf[br * BLK + r, pl.ds(v * L, L)]
                          for r in range(BLK)]
                    while len(xs) > 1:
                        xs = [xs[i] + xs[i + 1] for i in range(0, len(xs), 2)]
                    sums.append(xs[0])
                for dv, s in enumerate(sums):
                    rowsum[pl.ds((v0 + dv) * L, L)] = s
            br_glob = chunk * BR_PER_CHUNK + br
            orow = br_glob // 2
            ocol0 = (br_glob % 2) * W
            for g in range(W // L):
                idx0 = lanes * BLK + g * (L * BLK)
                gs = [plsc.load_gather(rowsum, [idx0 + j]) for j in range(BLK)]
                while len(gs) > 1:
                    gs = [gs[i] + gs[i + 1] for i in range(0, len(gs), 2)]
                acc = gs[0]
                cls = (acc >= THRESH[0]).astype(jnp.int32)
                for t in THRESH[1:]:
                    cls = cls + (acc >= t).astype(jnp.int32)
                out_buf[orow, pl.ds(ocol0 + g * L, L)] = cls
            return c2

        lax.fori_loop(0, BR_PER_CHUNK, br_body, 0)

    pltpu.async_copy(src_rows(0), bufs[0], sems[0])
    pltpu.async_copy(src_rows(1), bufs[1], sems[1])

    def chunk_pair(i, carry):
        for b in range(2):
            chunk = i * 2 + b
            pltpu.make_async_copy(src_rows(0), bufs[b], sems[b]).wait()
            compute_chunk(chunk, bufs[b])
            pltpu.async_copy(src_rows(chunk + 2), bufs[b], sems[b])
        return carry

    lax.fori_loop(0, NCHUNK // 2 - 1, chunk_pair, 0)
    for b in range(2):
        pltpu.make_async_copy(src_rows(0), bufs[b], sems[b]).wait()
        compute_chunk(NCHUNK - 2 + b, bufs[b])
    pltpu.sync_copy(out_buf, cls_hbm.at[wid])


@jax.jit
def _sc_cls(gt4):
    mesh = plsc.VectorSubcoreMesh(core_axis_name="c", subcore_axis_name="s",
                                  num_cores=NC, num_subcores=NS)
    return pl.kernel(
        _sc_cls_body,
        out_type=jax.ShapeDtypeStruct((B, HW // 128, 128), jnp.int32),
        mesh=mesh,
        compiler_params=pltpu.CompilerParams(needs_layout_passes=False),
        scratch_types=[
            pltpu.VMEM((CHUNK, GW), jnp.int32),
            pltpu.VMEM((CHUNK, GW), jnp.int32),
            pltpu.VMEM((GW,), jnp.int32),
            pltpu.VMEM((HW // 128, 128), jnp.int32),
            pltpu.SemaphoreType.DMA,
            pltpu.SemaphoreType.DMA,
        ],
    )(gt4)


IB = 8
PR = HW // 128


def _tc_loss_body(logits_ref, cls_ref, out_ref):
    x = logits_ref[...].reshape(IB, C, PR, 128)
    s = jnp.sum(jnp.exp(x), axis=1, keepdims=True)
    lse = jnp.log(s)
    cls = cls_ref[...][:, None, :, :]
    tgt = cls - 1
    picked = jnp.zeros_like(lse)
    for cc in range(C):
        picked = picked + jnp.where(tgt == cc, x[:, cc:cc + 1, :, :], 0.0)
    contrib = jnp.sum(jnp.where(cls > 0, lse - picked, 0.0))

    @pl.when(pl.program_id(0) == 0)
    def _():
        out_ref[0, 0] = 0.0

    out_ref[0, 0] += contrib


@jax.jit
def _tc_loss(logits4, cls3):
    return pl.pallas_call(
        _tc_loss_body,
        grid=(B // IB,),
        in_specs=[
            pl.BlockSpec((IB, C * PR, 128), lambda b: (b, 0, 0)),
            pl.BlockSpec((IB, PR, 128), lambda b: (b, 0, 0)),
        ],
        out_specs=pl.BlockSpec((1, 1), lambda b: (0, 0),
                               memory_space=pltpu.SMEM),
        out_shape=jax.ShapeDtypeStruct((1, 1), jnp.float32),
    )(logits4, cls3)


def kernel(logit_maps, gt_den_maps):
    cls = _sc_cls(gt_den_maps)
    logits_lin = logit_maps.reshape(B, C * PR, 128)
    total = _tc_loss(logits_lin, cls)
    loss = total[0, 0] * jnp.float32(1.0 / B)
    return (loss, {"cls_zice": lax.stop_gradient(loss)})
